# baseline (speedup 1.0000x reference)
import functools

import jax
import jax.numpy as jnp
from jax import lax
from jax.experimental import pallas as pl
from jax.experimental.pallas import tpu as pltpu
from jax.experimental.pallas import tpu_sc as plsc

N = 10000
CH = 128
E = 320000
NW = 32
EPW = E // NW
BA = 125
KA = EPW // BA
BD = 128
KD = 80
EPAD = KD * BD - EPW
RPT = 624
NPAD = 10240
RPD = NPAD // 16

_mesh = plsc.VectorSubcoreMesh(
    core_axis_name="c", subcore_axis_name="s", num_cores=2, num_subcores=16)

_f32 = jnp.float32


@functools.partial(
    pl.kernel,
    out_type=jax.ShapeDtypeStruct((2, NPAD), _f32),
    mesh=_mesh,
    scratch_types=[
        pltpu.VMEM_SHARED((NPAD,), _f32),
        pltpu.VMEM((KD, BD), jnp.int32),
        pltpu.VMEM((BD,), _f32),
        pltpu.VMEM((RPD,), _f32),
        pltpu.SemaphoreType.DMA,
    ],
)
def _deg_kernel(dst_hbm, hist_hbm, hist_sp, idx_v, ones_v, buf_v, semw):
    c = lax.axis_index("c")
    s = lax.axis_index("s")
    for i in range(RPD // 16):
        buf_v[pl.ds(16 * i, 16)] = jnp.zeros((16,), _f32)
    pltpu.sync_copy(buf_v, hist_sp.at[pl.ds(RPD * s, RPD)])
    for i in range(BD // 16):
        ones_v[pl.ds(16 * i, 16)] = jnp.ones((16,), _f32)
    pltpu.sync_copy(dst_hbm.at[c * 16 + s], idx_v)
    plsc.subcore_barrier()

    def body(k, carry):
        pltpu.async_copy(ones_v, hist_sp.at[idx_v.at[k]], semw, add=True)
        return carry

    lax.fori_loop(0, KD, body, 0)

    def drain(k, carry):
        pltpu.make_async_copy(ones_v, hist_sp.at[idx_v.at[0]], semw).wait()
        return carry

    lax.fori_loop(0, KD, drain, 0)
    plsc.subcore_barrier()

    @pl.when(s == 0)
    def _():
        pltpu.sync_copy(hist_sp, hist_hbm.at[c])


@functools.partial(
    pl.kernel,
    out_type=jax.ShapeDtypeStruct((2, N, CH), _f32),
    mesh=_mesh,
    scratch_types=[
        pltpu.VMEM_SHARED((N, CH), _f32),
        pltpu.VMEM((KA, BA), jnp.int32),
        pltpu.VMEM((BA,), jnp.int32),
        pltpu.VMEM((BA,), jnp.int32),
        pltpu.VMEM((BA, CH), _f32),
        pltpu.VMEM((BA, CH), _f32),
        pltpu.SemaphoreType.DMA,
        pltpu.SemaphoreType.DMA,
        pltpu.SemaphoreType.DMA,
        pltpu.SemaphoreType.DMA,
        pltpu.SemaphoreType.DMA,
        pltpu.SemaphoreType.DMA,
    ],
)
def _agg_kernel(z_hbm, src_hbm, dst_hbm, out_hbm,
                acc_sp, dstb, sb0, sb1, rows_v0, rows_v1,
                semi0, semi1, semr0, semr1, semw0, semw1):
    c = lax.axis_index("c")
    s = lax.axis_index("s")
    w = c * 16 + s
    row0 = pl.multiple_of(RPT * s, 8)
    pltpu.sync_copy(z_hbm.at[pl.ds(row0, RPT)], acc_sp.at[pl.ds(row0, RPT)])

    @pl.when(s == 0)
    def _():
        pltpu.sync_copy(z_hbm.at[pl.ds(16 * RPT, N - 16 * RPT)],
                        acc_sp.at[pl.ds(16 * RPT, N - 16 * RPT)])

    def load_src(k, sb, semi):
        pltpu.async_copy(src_hbm.at[w, k], sb, semi)

    def wait_src(sb, semi):
        pltpu.make_async_copy(src_hbm.at[w, 0], sb, semi).wait()

    def gather(sb, buf, semr):
        pltpu.async_copy(z_hbm.at[sb], buf, semr)

    def wait_rows(buf, semr):
        pltpu.make_async_copy(z_hbm.at[sb0], buf, semr).wait()

    def scatter(k, buf, semw):
        pltpu.async_copy(buf, acc_sp.at[dstb.at[k]], semw, add=True)

    def wait_scat(buf, semw):
        pltpu.make_async_copy(buf, acc_sp.at[dstb.at[0]], semw).wait()

    pltpu.sync_copy(dst_hbm.at[w], dstb)
    pltpu.sync_copy(src_hbm.at[w, 0], sb0)
    plsc.subcore_barrier()
    gather(sb0, rows_v0, semr0)
    load_src(1, sb1, semi1)
    wait_rows(rows_v0, semr0)
    load_src(2, sb0, semi0)
    scatter(0, rows_v0, semw0)
    wait_src(sb1, semi1)
    gather(sb1, rows_v1, semr1)
    wait_rows(rows_v1, semr1)
    load_src(3, sb1, semi1)
    scatter(1, rows_v1, semw1)
    wait_src(sb0, semi0)
    wait_scat(rows_v0, semw0)
    gather(sb0, rows_v0, semr0)

    def body(j, carry):
        ka = 2 * j
        wait_rows(rows_v0, semr0)
        load_src(ka + 2, sb0, semi0)
        scatter(ka, rows_v0, semw0)
        wait_src(sb1, semi1)
        wait_scat(rows_v1, semw1)
        gather(sb1, rows_v1, semr1)
        wait_rows(rows_v1, semr1)
        load_src(ka + 3, sb1, semi1)
        scatter(ka + 1, rows_v1, semw1)
        wait_src(sb0, semi0)
        wait_scat(rows_v0, semw0)
        gather(sb0, rows_v0, semr0)
        return carry

    lax.fori_loop(1, KA // 2 - 1, body, 0)
    wait_rows(rows_v0, semr0)
    scatter(KA - 2, rows_v0, semw0)
    wait_src(sb1, semi1)
    wait_scat(rows_v1, semw1)
    gather(sb1, rows_v1, semr1)
    wait_rows(rows_v1, semr1)
    scatter(KA - 1, rows_v1, semw1)
    wait_scat(rows_v0, semw0)
    wait_scat(rows_v1, semw1)
    plsc.subcore_barrier()
    pltpu.sync_copy(acc_sp.at[pl.ds(row0, RPT)],
                    out_hbm.at[c, pl.ds(row0, RPT)])

    @pl.when(s == 0)
    def _():
        pltpu.sync_copy(acc_sp.at[pl.ds(16 * RPT, N - 16 * RPT)],
                        out_hbm.at[c, pl.ds(16 * RPT, N - 16 * RPT)])


def _mm_body(x_ref, w_ref, h0_ref, h1_ref, z_ref):
    d = lax.rsqrt(h0_ref[...] + h1_ref[...] + 1.0)
    z_ref[...] = jnp.dot(x_ref[...] * d, w_ref[...],
                         preferred_element_type=_f32)


def _fin_body(a0_ref, a1_ref, z_ref, h0_ref, h1_ref, b_ref, o_ref):
    d = lax.rsqrt(h0_ref[...] + h1_ref[...] + 1.0)
    t = d * (a0_ref[...] + a1_ref[...] - z_ref[...]) + b_ref[...]
    o_ref[...] = jnp.where(t >= 0.0, 2.0 * t, 1.01 * t)


_BM = 400


def _mm(x, W, h0c, h1c):
    return pl.pallas_call(
        _mm_body,
        grid=(N // _BM,),
        in_specs=[
            pl.BlockSpec((_BM, CH), lambda i: (i, 0)),
            pl.BlockSpec((CH, CH), lambda i: (0, 0)),
            pl.BlockSpec((_BM, 1), lambda i: (i, 0)),
            pl.BlockSpec((_BM, 1), lambda i: (i, 0)),
        ],
        out_specs=pl.BlockSpec((_BM, CH), lambda i: (i, 0)),
        out_shape=jax.ShapeDtypeStruct((N, CH), _f32),
    )(x, W, h0c, h1c)


def _fin(a0, a1, z, h0c, h1c, b2):
    return pl.pallas_call(
        _fin_body,
        grid=(N // _BM,),
        in_specs=[
            pl.BlockSpec((_BM, CH), lambda i: (i, 0)),
            pl.BlockSpec((_BM, CH), lambda i: (i, 0)),
            pl.BlockSpec((_BM, CH), lambda i: (i, 0)),
            pl.BlockSpec((_BM, 1), lambda i: (i, 0)),
            pl.BlockSpec((_BM, 1), lambda i: (i, 0)),
            pl.BlockSpec((1, CH), lambda i: (0, 0)),
        ],
        out_specs=pl.BlockSpec((_BM, CH), lambda i: (i, 0)),
        out_shape=jax.ShapeDtypeStruct((N, CH), _f32),
    )(a0, a1, z, h0c, h1c, b2)


@jax.jit
def kernel(x, edges, W, b):
    src = edges[0].astype(jnp.int32).reshape(NW, KA, BA)
    dst = edges[1].astype(jnp.int32).reshape(NW, KA, BA)
    pad = jnp.broadcast_to(N + jnp.arange(EPAD, dtype=jnp.int32), (NW, EPAD))
    dst_deg = jnp.concatenate(
        [edges[1].astype(jnp.int32).reshape(NW, EPW), pad], axis=1
    ).reshape(NW, KD, BD)
    hist = _deg_kernel(dst_deg)
    h0c = hist[0, :N][:, None]
    h1c = hist[1, :N][:, None]
    z = _mm(x, W, h0c, h1c)
    acc = _agg_kernel(z, src, dst)
    return _fin(acc[0], acc[1], z, h0c, h1c, b[None, :])


# speedup vs baseline: 1.0038x; 1.0038x over previous
import functools

import jax
import jax.numpy as jnp
from jax import lax
from jax.experimental import pallas as pl
from jax.experimental.pallas import tpu as pltpu
from jax.experimental.pallas import tpu_sc as plsc

N = 10000
CH = 128
E = 320000
NW = 32
EPW = E // NW
BA = 125
KA = EPW // BA
BD = 128
KD = 80
EPAD = KD * BD - EPW
RPT = 624
NPAD = 10240
RPD = NPAD // 16

_mesh = plsc.VectorSubcoreMesh(
    core_axis_name="c", subcore_axis_name="s", num_cores=2, num_subcores=16)

_f32 = jnp.float32


@functools.partial(
    pl.kernel,
    out_type=jax.ShapeDtypeStruct((2, NPAD), _f32),
    mesh=_mesh,
    scratch_types=[
        pltpu.VMEM_SHARED((NPAD,), _f32),
        pltpu.VMEM((KD, BD), jnp.int32),
        pltpu.VMEM((BD,), _f32),
        pltpu.VMEM((RPD,), _f32),
        pltpu.SemaphoreType.DMA,
    ],
)
def _deg_kernel(dst_hbm, hist_hbm, hist_sp, idx_v, ones_v, buf_v, semw):
    c = lax.axis_index("c")
    s = lax.axis_index("s")
    for i in range(RPD // 16):
        buf_v[pl.ds(16 * i, 16)] = jnp.zeros((16,), _f32)
    pltpu.sync_copy(buf_v, hist_sp.at[pl.ds(RPD * s, RPD)])
    for i in range(BD // 16):
        ones_v[pl.ds(16 * i, 16)] = jnp.ones((16,), _f32)
    pltpu.sync_copy(dst_hbm.at[c * 16 + s], idx_v)
    plsc.subcore_barrier()

    def body(k, carry):
        pltpu.async_copy(ones_v, hist_sp.at[idx_v.at[k]], semw, add=True)
        return carry

    lax.fori_loop(0, KD, body, 0)

    def drain(k, carry):
        pltpu.make_async_copy(ones_v, hist_sp.at[idx_v.at[0]], semw).wait()
        return carry

    lax.fori_loop(0, KD, drain, 0)
    plsc.subcore_barrier()

    @pl.when(s == 0)
    def _():
        pltpu.sync_copy(hist_sp, hist_hbm.at[c])


@functools.partial(
    pl.kernel,
    out_type=jax.ShapeDtypeStruct((2, N, CH), _f32),
    mesh=_mesh,
    scratch_types=[
        pltpu.VMEM_SHARED((N, CH), _f32),
        pltpu.VMEM((KA, BA), jnp.int32),
        pltpu.VMEM((BA,), jnp.int32),
        pltpu.VMEM((BA,), jnp.int32),
        pltpu.VMEM((BA, CH), _f32),
        pltpu.VMEM((BA, CH), _f32),
        pltpu.SemaphoreType.DMA,
        pltpu.SemaphoreType.DMA,
        pltpu.SemaphoreType.DMA,
        pltpu.SemaphoreType.DMA,
        pltpu.SemaphoreType.DMA,
        pltpu.SemaphoreType.DMA,
    ],
)
def _agg_kernel(z_hbm, src_hbm, dst_hbm, out_hbm,
                acc_sp, dstb, sb0, sb1, rows_v0, rows_v1,
                semi0, semi1, semr0, semr1, semw0, semw1):
    c = lax.axis_index("c")
    s = lax.axis_index("s")
    w = c * 16 + s
    row0 = pl.multiple_of(RPT * s, 8)
    pltpu.sync_copy(z_hbm.at[pl.ds(row0, RPT)], acc_sp.at[pl.ds(row0, RPT)])

    @pl.when(s == 0)
    def _():
        pltpu.sync_copy(z_hbm.at[pl.ds(16 * RPT, N - 16 * RPT)],
                        acc_sp.at[pl.ds(16 * RPT, N - 16 * RPT)])

    def load_src(k, sb, semi):
        pltpu.async_copy(src_hbm.at[w, k], sb, semi)

    def wait_src(sb, semi):
        pltpu.make_async_copy(src_hbm.at[w, 0], sb, semi).wait()

    def gather(sb, buf, semr):
        pltpu.async_copy(z_hbm.at[sb], buf, semr)

    def wait_rows(buf, semr):
        pltpu.make_async_copy(z_hbm.at[sb0], buf, semr).wait()

    def scatter(k, buf, semw):
        pltpu.async_copy(buf, acc_sp.at[dstb.at[k]], semw, add=True)

    def wait_scat(buf, semw):
        pltpu.make_async_copy(buf, acc_sp.at[dstb.at[0]], semw).wait()

    pltpu.sync_copy(dst_hbm.at[w], dstb)
    pltpu.sync_copy(src_hbm.at[w, 0], sb0)
    plsc.subcore_barrier()
    gather(sb0, rows_v0, semr0)
    load_src(1, sb1, semi1)
    wait_rows(rows_v0, semr0)
    load_src(2, sb0, semi0)
    scatter(0, rows_v0, semw0)
    wait_src(sb1, semi1)
    gather(sb1, rows_v1, semr1)
    wait_rows(rows_v1, semr1)
    load_src(3, sb1, semi1)
    scatter(1, rows_v1, semw1)
    wait_src(sb0, semi0)
    wait_scat(rows_v0, semw0)
    gather(sb0, rows_v0, semr0)

    def body(j, carry):
        ka = 2 * j
        wait_rows(rows_v0, semr0)
        load_src(ka + 2, sb0, semi0)
        scatter(ka, rows_v0, semw0)
        wait_src(sb1, semi1)
        wait_scat(rows_v1, semw1)
        gather(sb1, rows_v1, semr1)
        wait_rows(rows_v1, semr1)
        load_src(ka + 3, sb1, semi1)
        scatter(ka + 1, rows_v1, semw1)
        wait_src(sb0, semi0)
        wait_scat(rows_v0, semw0)
        gather(sb0, rows_v0, semr0)
        return carry

    lax.fori_loop(1, KA // 2 - 1, body, 0)
    wait_rows(rows_v0, semr0)
    scatter(KA - 2, rows_v0, semw0)
    wait_src(sb1, semi1)
    wait_scat(rows_v1, semw1)
    gather(sb1, rows_v1, semr1)
    wait_rows(rows_v1, semr1)
    scatter(KA - 1, rows_v1, semw1)
    wait_scat(rows_v0, semw0)
    wait_scat(rows_v1, semw1)
    plsc.subcore_barrier()
    pltpu.sync_copy(acc_sp.at[pl.ds(row0, RPT)],
                    out_hbm.at[c, pl.ds(row0, RPT)])

    @pl.when(s == 0)
    def _():
        pltpu.sync_copy(acc_sp.at[pl.ds(16 * RPT, N - 16 * RPT)],
                        out_hbm.at[c, pl.ds(16 * RPT, N - 16 * RPT)])


def _mm_body(x_ref, w_ref, h0_ref, h1_ref, z_ref):
    d = lax.rsqrt(h0_ref[...] + h1_ref[...] + 1.0)
    z_ref[...] = jnp.dot(x_ref[...], w_ref[...],
                         preferred_element_type=_f32) * d


def _fin_body(a0_ref, a1_ref, z_ref, h0_ref, h1_ref, b_ref, o_ref):
    d = lax.rsqrt(h0_ref[...] + h1_ref[...] + 1.0)
    t = d * (a0_ref[...] + a1_ref[...] - z_ref[...]) + b_ref[...]
    o_ref[...] = jnp.where(t >= 0.0, 2.0 * t, 1.01 * t)


_BM = 400


def _mm(x, W, h0c, h1c):
    return pl.pallas_call(
        _mm_body,
        grid=(N // _BM,),
        in_specs=[
            pl.BlockSpec((_BM, CH), lambda i: (i, 0)),
            pl.BlockSpec((CH, CH), lambda i: (0, 0)),
            pl.BlockSpec((_BM, 1), lambda i: (i, 0)),
            pl.BlockSpec((_BM, 1), lambda i: (i, 0)),
        ],
        out_specs=pl.BlockSpec((_BM, CH), lambda i: (i, 0)),
        out_shape=jax.ShapeDtypeStruct((N, CH), _f32),
    )(x, W, h0c, h1c)


def _fin(a0, a1, z, h0c, h1c, b2):
    return pl.pallas_call(
        _fin_body,
        grid=(N // _BM,),
        in_specs=[
            pl.BlockSpec((_BM, CH), lambda i: (i, 0)),
            pl.BlockSpec((_BM, CH), lambda i: (i, 0)),
            pl.BlockSpec((_BM, CH), lambda i: (i, 0)),
            pl.BlockSpec((_BM, 1), lambda i: (i, 0)),
            pl.BlockSpec((_BM, 1), lambda i: (i, 0)),
            pl.BlockSpec((1, CH), lambda i: (0, 0)),
        ],
        out_specs=pl.BlockSpec((_BM, CH), lambda i: (i, 0)),
        out_shape=jax.ShapeDtypeStruct((N, CH), _f32),
    )(a0, a1, z, h0c, h1c, b2)


@jax.jit
def kernel(x, edges, W, b):
    src = edges[0].astype(jnp.int32).reshape(NW, KA, BA)
    dst = edges[1].astype(jnp.int32).reshape(NW, KA, BA)
    pad = jnp.broadcast_to(N + jnp.arange(EPAD, dtype=jnp.int32), (NW, EPAD))
    dst_deg = jnp.concatenate(
        [edges[1].astype(jnp.int32).reshape(NW, EPW), pad], axis=1
    ).reshape(NW, KD, BD)
    hist = _deg_kernel(dst_deg)
    h0c = hist[0, :N][:, None]
    h1c = hist[1, :N][:, None]
    z = _mm(x, W, h0c, h1c)
    acc = _agg_kernel(z, src, dst)
    return _fin(acc[0], acc[1], z, h0c, h1c, b[None, :])


# speedup vs baseline: 1.0241x; 1.0203x over previous
import functools

import jax
import jax.numpy as jnp
from jax import lax
from jax.experimental import pallas as pl
from jax.experimental.pallas import tpu as pltpu
from jax.experimental.pallas import tpu_sc as plsc

N = 10000
CH = 128
E = 320000
NW = 32
EPW = E // NW
BA = 125
KA = EPW // BA
BD = 128
KD = 80
EPAD = KD * BD - EPW
RPT = 624
NPAD = 10240
RPD = NPAD // 16

_mesh = plsc.VectorSubcoreMesh(
    core_axis_name="c", subcore_axis_name="s", num_cores=2, num_subcores=16)

_f32 = jnp.float32


@functools.partial(
    pl.kernel,
    out_type=jax.ShapeDtypeStruct((2, NPAD), _f32),
    mesh=_mesh,
    scratch_types=[
        pltpu.VMEM_SHARED((NPAD,), _f32),
        pltpu.VMEM((KD, BD), jnp.int32),
        pltpu.VMEM((BD,), _f32),
        pltpu.VMEM((RPD,), _f32),
        pltpu.SemaphoreType.DMA,
    ],
)
def _deg_kernel(dst_hbm, hist_hbm, hist_sp, idx_v, ones_v, buf_v, semw):
    c = lax.axis_index("c")
    s = lax.axis_index("s")
    for i in range(RPD // 16):
        buf_v[pl.ds(16 * i, 16)] = jnp.zeros((16,), _f32)
    pltpu.sync_copy(buf_v, hist_sp.at[pl.ds(RPD * s, RPD)])
    for i in range(BD // 16):
        ones_v[pl.ds(16 * i, 16)] = jnp.ones((16,), _f32)
    pltpu.sync_copy(dst_hbm.at[c * 16 + s], idx_v)
    plsc.subcore_barrier()

    def body(k, carry):
        pltpu.async_copy(ones_v, hist_sp.at[idx_v.at[k]], semw, add=True)
        return carry

    lax.fori_loop(0, KD, body, 0)

    def drain(k, carry):
        pltpu.make_async_copy(ones_v, hist_sp.at[idx_v.at[0]], semw).wait()
        return carry

    lax.fori_loop(0, KD, drain, 0)
    plsc.subcore_barrier()

    @pl.when(s == 0)
    def _():
        pltpu.sync_copy(hist_sp, hist_hbm.at[c])


@functools.partial(
    pl.kernel,
    out_type=jax.ShapeDtypeStruct((2, N, CH), _f32),
    mesh=_mesh,
    scratch_types=[
        pltpu.VMEM_SHARED((N, CH), _f32),
        pltpu.VMEM((2, BA), jnp.int32),
        pltpu.VMEM((2, BA), jnp.int32),
        pltpu.VMEM((BA, CH), _f32),
        pltpu.VMEM((BA, CH), _f32),
        pltpu.SemaphoreType.DMA,
        pltpu.SemaphoreType.DMA,
        pltpu.SemaphoreType.DMA,
        pltpu.SemaphoreType.DMA,
    ],
)
def _agg_kernel(z_hbm, idx_hbm, out_hbm,
                acc_sp, ib0, ib1, rows_v0, rows_v1,
                semi0, semi1, semr0, semr1):
    c = lax.axis_index("c")
    s = lax.axis_index("s")
    w = c * 16 + s
    row0 = pl.multiple_of(RPT * s, 8)
    pltpu.sync_copy(z_hbm.at[pl.ds(row0, RPT)], acc_sp.at[pl.ds(row0, RPT)])

    @pl.when(s == 0)
    def _():
        pltpu.sync_copy(z_hbm.at[pl.ds(16 * RPT, N - 16 * RPT)],
                        acc_sp.at[pl.ds(16 * RPT, N - 16 * RPT)])

    def load_idx(k, ib, semi):
        pltpu.async_copy(idx_hbm.at[w, k], ib, semi)

    def wait_idx(ib, semi):
        pltpu.make_async_copy(idx_hbm.at[w, 0], ib, semi).wait()

    def gather(ib, buf, semr):
        pltpu.async_copy(z_hbm.at[ib.at[0]], buf, semr)

    def wait_rows(buf, semr):
        pltpu.make_async_copy(z_hbm.at[ib0.at[0]], buf, semr).wait()

    def scatter(ib, buf):
        pltpu.sync_copy(buf, acc_sp.at[ib.at[1]], add=True)

    pltpu.sync_copy(idx_hbm.at[w, 0], ib0)
    plsc.subcore_barrier()
    gather(ib0, rows_v0, semr0)
    load_idx(1, ib1, semi1)

    def body(j, carry):
        k = 2 * j
        wait_idx(ib1, semi1)
        gather(ib1, rows_v1, semr1)
        wait_rows(rows_v0, semr0)
        scatter(ib0, rows_v0)
        load_idx(k + 2, ib0, semi0)
        wait_idx(ib0, semi0)
        gather(ib0, rows_v0, semr0)
        wait_rows(rows_v1, semr1)
        scatter(ib1, rows_v1)
        load_idx(k + 3, ib1, semi1)
        return carry

    lax.fori_loop(0, (KA - 2) // 2, body, 0)
    wait_idx(ib1, semi1)
    gather(ib1, rows_v1, semr1)
    wait_rows(rows_v0, semr0)
    scatter(ib0, rows_v0)
    wait_rows(rows_v1, semr1)
    scatter(ib1, rows_v1)
    plsc.subcore_barrier()
    pltpu.sync_copy(acc_sp.at[pl.ds(row0, RPT)],
                    out_hbm.at[c, pl.ds(row0, RPT)])

    @pl.when(s == 0)
    def _():
        pltpu.sync_copy(acc_sp.at[pl.ds(16 * RPT, N - 16 * RPT)],
                        out_hbm.at[c, pl.ds(16 * RPT, N - 16 * RPT)])


def _mm_body(x_ref, w_ref, h0_ref, h1_ref, z_ref):
    d = lax.rsqrt(h0_ref[...] + h1_ref[...] + 1.0)
    z_ref[...] = jnp.dot(x_ref[...], w_ref[...],
                         preferred_element_type=_f32) * d


def _fin_body(a0_ref, a1_ref, z_ref, h0_ref, h1_ref, b_ref, o_ref):
    d = lax.rsqrt(h0_ref[...] + h1_ref[...] + 1.0)
    t = d * (a0_ref[...] + a1_ref[...] - z_ref[...]) + b_ref[...]
    o_ref[...] = jnp.where(t >= 0.0, 2.0 * t, 1.01 * t)


_BM = 400


def _mm(x, W, h0c, h1c):
    return pl.pallas_call(
        _mm_body,
        grid=(N // _BM,),
        in_specs=[
            pl.BlockSpec((_BM, CH), lambda i: (i, 0)),
            pl.BlockSpec((CH, CH), lambda i: (0, 0)),
            pl.BlockSpec((_BM, 1), lambda i: (i, 0)),
            pl.BlockSpec((_BM, 1), lambda i: (i, 0)),
        ],
        out_specs=pl.BlockSpec((_BM, CH), lambda i: (i, 0)),
        out_shape=jax.ShapeDtypeStruct((N, CH), _f32),
    )(x, W, h0c, h1c)


def _fin(a0, a1, z, h0c, h1c, b2):
    return pl.pallas_call(
        _fin_body,
        grid=(N // _BM,),
        in_specs=[
            pl.BlockSpec((_BM, CH), lambda i: (i, 0)),
            pl.BlockSpec((_BM, CH), lambda i: (i, 0)),
            pl.BlockSpec((_BM, CH), lambda i: (i, 0)),
            pl.BlockSpec((_BM, 1), lambda i: (i, 0)),
            pl.BlockSpec((_BM, 1), lambda i: (i, 0)),
            pl.BlockSpec((1, CH), lambda i: (0, 0)),
        ],
        out_specs=pl.BlockSpec((_BM, CH), lambda i: (i, 0)),
        out_shape=jax.ShapeDtypeStruct((N, CH), _f32),
    )(a0, a1, z, h0c, h1c, b2)


@jax.jit
def kernel(x, edges, W, b):
    src = edges[0].astype(jnp.int32).reshape(NW, KA, BA)
    dst = edges[1].astype(jnp.int32).reshape(NW, KA, BA)
    idx = jnp.stack([src, dst], axis=2)
    pad = jnp.broadcast_to(N + jnp.arange(EPAD, dtype=jnp.int32), (NW, EPAD))
    dst_deg = jnp.concatenate(
        [edges[1].astype(jnp.int32).reshape(NW, EPW), pad], axis=1
    ).reshape(NW, KD, BD)
    hist = _deg_kernel(dst_deg)
    h0c = hist[0, :N][:, None]
    h1c = hist[1, :N][:, None]
    z = _mm(x, W, h0c, h1c)
    acc = _agg_kernel(z, idx)
    return _fin(acc[0], acc[1], z, h0c, h1c, b[None, :])
